# baseline (speedup 1.0000x reference)
import functools

import jax
import jax.numpy as jnp
from jax import lax
from jax.experimental import pallas as pl
from jax.experimental.pallas import tpu as pltpu
from jax.experimental.pallas import tpu_sc as plsc

NC = 2
NS = 16
L = 16
NW = NC * NS
B = 16384
D = 64
BPW = B // NW
NCH = BPW // 128
HALF = BPW // 2
CPH = HALF // L
NUM_HALF_ROWS = 100000 * D // 128


def _svdpp_body(sids_h, pids_h, sfac_h, pfac_h, sbias_h, pbias_h, g_h, out_h,
                sidx_v, pidx_v, sdidx_v, pdidx_v, sh_v, ph_v,
                srows_v, prows_v, sb_v, pb_v, g_v, out_v, sem, semb):
    w = lax.axis_index("s") * NC + lax.axis_index("c")
    base = w * BPW

    pltpu.sync_copy(sids_h.at[pl.ds(w * NCH, NCH)], sidx_v)
    pltpu.sync_copy(pids_h.at[pl.ds(w * NCH, NCH)], pidx_v)
    pltpu.sync_copy(g_h, g_v)

    bcps = []
    for c in range(NCH):
        r = pl.ds(c * 128, 128)
        bcps.append(pltpu.async_copy(sbias_h.at[sidx_v.at[c]], sb_v.at[r], semb))
        bcps.append(pltpu.async_copy(pbias_h.at[pidx_v.at[c]], pb_v.at[r], semb))

    for c in range(NCH):
        for j in range(128 // L):
            sl = pl.ds(j * L, L)
            fl = pl.ds(c * 128 + j * L, L)
            sv = sidx_v[c, sl]
            pv = pidx_v[c, sl]
            sdidx_v[c, sl] = lax.shift_right_logical(sv, 1)
            pdidx_v[c, sl] = lax.shift_right_logical(pv, 1)
            sh_v[fl] = lax.shift_left(lax.bitwise_and(sv, 1), 6)
            ph_v[fl] = lax.shift_left(lax.bitwise_and(pv, 1), 6)

    iota = lax.broadcasted_iota(jnp.int32, (L,), 0)
    ones = jnp.ones((L,), jnp.int32)
    gvec = g_v[...]

    for h in range(2):
        cps = []
        for cc in range(2):
            c = 2 * h + cc
            r = pl.ds(cc * 128, 128)
            cps.append(pltpu.async_copy(sfac_h.at[sdidx_v.at[c]],
                                        srows_v.at[r], sem))
            cps.append(pltpu.async_copy(pfac_h.at[pdidx_v.at[c]],
                                        prows_v.at[r], sem))
        for cp in cps:
            cp.wait()
        if h == 0:
            for cp in bcps:
                cp.wait()

        def chunk(i, carry, h=h):
            lrow = (i - h * CPH) * L + iota
            sh0 = sh_v[pl.ds(i * L, L)]
            ph0 = ph_v[pl.ds(i * L, L)]

            def dbody(d, st):
                acc, scol, pcol = st
                sv = plsc.load_gather(srows_v, [lrow, scol])
                pv = plsc.load_gather(prows_v, [lrow, pcol])
                return acc + sv * pv, scol + ones, pcol + ones

            acc, _, _ = lax.fori_loop(
                0, D, dbody,
                (jnp.zeros((L,), jnp.float32), sh0, ph0), unroll=8)
            sl = pl.ds(i * L, L)
            out_v[sl] = acc + sb_v[sl] + pb_v[sl] + gvec
            return carry

        lax.fori_loop(h * CPH, (h + 1) * CPH, chunk, 0)

    pltpu.sync_copy(out_v, out_h.at[pl.ds(base, BPW)])


_svdpp = functools.partial(
    pl.kernel,
    out_type=jax.ShapeDtypeStruct((B,), jnp.float32),
    mesh=plsc.VectorSubcoreMesh(core_axis_name="c", subcore_axis_name="s"),
    scratch_types=[
        pltpu.VMEM((NCH, 128), jnp.int32),
        pltpu.VMEM((NCH, 128), jnp.int32),
        pltpu.VMEM((NCH, 128), jnp.int32),
        pltpu.VMEM((NCH, 128), jnp.int32),
        pltpu.VMEM((BPW,), jnp.int32),
        pltpu.VMEM((BPW,), jnp.int32),
        pltpu.VMEM((HALF, 128), jnp.float32),
        pltpu.VMEM((HALF, 128), jnp.float32),
        pltpu.VMEM((BPW,), jnp.float32),
        pltpu.VMEM((BPW,), jnp.float32),
        pltpu.VMEM((L,), jnp.float32),
        pltpu.VMEM((BPW,), jnp.float32),
        pltpu.SemaphoreType.DMA,
        pltpu.SemaphoreType.DMA,
    ],
    compiler_params=pltpu.CompilerParams(needs_layout_passes=False,
                                         use_tc_tiling_on_sc=True),
)(_svdpp_body)


def kernel(scientist_ids, paper_ids, scientist_factors, paper_factors,
           implicit_factors, scientist_bias, paper_bias, global_bias):
    del implicit_factors
    sids = scientist_ids.astype(jnp.int32).reshape(NW * NCH, 128)
    pids = paper_ids.astype(jnp.int32).reshape(NW * NCH, 128)
    sfac = scientist_factors.reshape(NUM_HALF_ROWS, 128)
    pfac = paper_factors.reshape(NUM_HALF_ROWS, 128)
    sb = scientist_bias.reshape(-1)
    pb = paper_bias.reshape(-1)
    g16 = jnp.broadcast_to(global_bias.astype(jnp.float32).reshape(()), (L,))
    return _svdpp(sids, pids, sfac, pfac, sb, pb, g16)
